# baseline (speedup 1.0000x reference)
import jax
import jax.numpy as jnp
from jax import lax
from jax.experimental import pallas as pl
from jax.experimental.pallas import tpu as pltpu
from jax.experimental.pallas import tpu_sc as plsc

_NUM_EMB = 1000000
_DIM = 64
_SCALE = float(_DIM) ** 0.5

_NC, _NS = 2, 16
_NW = _NC * _NS
_BATCH, _SEQ = 16384, 50
_B = _BATCH * _SEQ
_NBLK = 7812
_W2ROWS = 500000
_G = 128
_NGRP = _B // _G
_GPW = _NGRP // _NW
_K1_IT = 246


def _splat(x):
    return lax.broadcast_in_dim(x, (16,), ())


def _k1_body(wt_hbm, w2_hbm, vin, vout, vtin, vtout, *sems):
    sg, ss = sems[:2], sems[2:]
    wid = lax.axis_index("s") * _NC + lax.axis_index("c")
    iota = lax.iota(jnp.int32, 16)
    drow = (iota >> 1)
    dcol = (iota & 1) << 6

    def blk_of(i):
        return jnp.minimum(wid + 32 * i, _NBLK - 1)

    def transpose_scale(src, dst, nebs):
        for eb in range(nebs):
            srccol = _splat(eb * 16) + iota
            dstrow = _splat(eb * 8) + drow
            for fb in range(4):
                fbs = _splat(fb * 16)

                @pl.loop(0, 16, unroll=8)
                def _d(d):
                    rot = (iota + _splat(d)) & 15
                    val = plsc.load_gather(src, [fbs + rot, srccol])
                    plsc.store_scatter(
                        dst, [dstrow, dcol + fbs + rot], val * _SCALE)

    for b in range(2):
        pltpu.async_copy(
            wt_hbm.at[:, pl.ds(blk_of(b) * 128, 128)], vin.at[b], sg[b])

    @pl.loop(0, _K1_IT, step=2)
    def _it(i):
        for b in range(2):
            pltpu.make_async_copy(
                wt_hbm.at[:, pl.ds(0, 128)], vin.at[b], sg[b]).wait()
            @pl.when(i > 0)
            def _():
                pltpu.make_async_copy(
                    vout.at[b], w2_hbm.at[pl.ds(0, 64)], ss[b]).wait()
            transpose_scale(vin.at[b], vout.at[b], 8)
            @pl.when(i < _K1_IT - 2)
            def _():
                pltpu.async_copy(
                    wt_hbm.at[:, pl.ds(blk_of(i + b + 2) * 128, 128)],
                    vin.at[b], sg[b])
            pltpu.async_copy(
                vout.at[b], w2_hbm.at[pl.ds(blk_of(i + b) * 64, 64)], ss[b])

    for b in range(2):
        pltpu.make_async_copy(
            vout.at[b], w2_hbm.at[pl.ds(0, 64)], ss[b]).wait()

    @pl.when(wid == 0)
    def _tail():
        pltpu.async_copy(
            wt_hbm.at[:, pl.ds(_NBLK * 128, 64)], vtin, sg[0]).wait()
        transpose_scale(vtin, vtout, 4)
        pltpu.async_copy(
            vtout, w2_hbm.at[pl.ds(_NBLK * 64, 32)], ss[0]).wait()


def _k2_body(idx_hbm, w2_hbm, out_hbm, rowidx, gb, sb, *sems):
    si, sg, ss = sems[0], sems[1:3], sems[3:]
    wid = lax.axis_index("s") * _NC + lax.axis_index("c")
    pltpu.async_copy(
        idx_hbm.at[pl.ds(wid * _GPW, _GPW)], rowidx, si).wait()

    iota = lax.iota(jnp.int32, 16)
    gg0 = wid * _GPW

    def transpose_out(b):
        for jb in range(8):
            rowv = _splat(jb * 16) + iota
            for fb in range(4):
                fbs = _splat(fb * 16)

                @pl.loop(0, 16, unroll=8)
                def _d(d):
                    rot = (iota + _splat(d)) & 15
                    val = plsc.load_gather(gb.at[b], [rowv, fbs + rot])
                    plsc.store_scatter(sb.at[b], [fbs + rot, rowv], val)

    for b in range(2):
        pltpu.async_copy(w2_hbm.at[rowidx.at[b]], gb.at[b], sg[b])

    @pl.loop(0, _GPW, step=2)
    def _grp(g0):
        for b in range(2):
            g = g0 + b
            pltpu.make_async_copy(
                w2_hbm.at[rowidx.at[b]], gb.at[b], sg[b]).wait()
            @pl.when(g0 > 0)
            def _():
                pltpu.make_async_copy(
                    sb.at[b], out_hbm.at[0, :, pl.ds(0, _G)], ss[b]).wait()
            transpose_out(b)
            @pl.when(g0 < _GPW - 2)
            def _():
                pltpu.async_copy(w2_hbm.at[rowidx.at[g + 2]], gb.at[b], sg[b])
            gg = gg0 + g
            pltpu.async_copy(
                sb.at[b],
                out_hbm.at[gg >> 7, :, pl.ds((gg & 127) * _G, _G)], ss[b])

    for b in range(2):
        pltpu.make_async_copy(
            sb.at[b], out_hbm.at[0, :, pl.ds(0, _G)], ss[b]).wait()


@jax.jit
def _emb(idx_flat, wt):
    mesh = plsc.VectorSubcoreMesh(core_axis_name="c", subcore_axis_name="s")
    w2 = pl.kernel(
        _k1_body,
        out_type=jax.ShapeDtypeStruct((_W2ROWS, 128), jnp.float32),
        mesh=mesh,
        compiler_params=pltpu.CompilerParams(needs_layout_passes=False),
        scratch_types=(
            [pltpu.VMEM((2, _DIM, 128), jnp.float32),
             pltpu.VMEM((2, _DIM, 128), jnp.float32),
             pltpu.VMEM((_DIM, 64), jnp.float32),
             pltpu.VMEM((32, 128), jnp.float32)]
            + [pltpu.SemaphoreType.DMA] * 4
        ),
    )(wt)
    out3 = pl.kernel(
        _k2_body,
        out_type=jax.ShapeDtypeStruct((_SEQ, _DIM, _BATCH), jnp.float32),
        mesh=mesh,
        compiler_params=pltpu.CompilerParams(
            needs_layout_passes=False, use_tc_tiling_on_sc=False),
        scratch_types=(
            [pltpu.VMEM((_GPW, _G), jnp.int32),
             pltpu.VMEM((2, _G, _DIM), jnp.float32),
             pltpu.VMEM((2, _DIM, _G), jnp.float32)]
            + [pltpu.SemaphoreType.DMA] * 5
        ),
    )(idx_flat.reshape(_NGRP, _G), w2.reshape(2 * _W2ROWS, _DIM))
    return out3


def kernel(input, weight):
    idx_flat = input.astype(jnp.int32).T.reshape(_B)
    out3 = _emb(idx_flat, weight.T)
    return out3.transpose(2, 0, 1)


# speedup vs baseline: 2.3988x; 2.3988x over previous
import jax
import jax.numpy as jnp
from jax import lax
from jax.experimental import pallas as pl
from jax.experimental.pallas import tpu as pltpu
from jax.experimental.pallas import tpu_sc as plsc

_NUM_EMB = 1000000
_DIM = 64
_SCALE = float(_DIM) ** 0.5

_NC, _NS = 2, 16
_NW = _NC * _NS
_BATCH, _SEQ = 16384, 50
_B = _BATCH * _SEQ
_NBLK = 7812
_W2ROWS = 500000
_G = 128
_NGRP = _B // _G
_GPW = _NGRP // _NW
_K1_IT = 246


def _splat(x):
    return lax.broadcast_in_dim(x, (16,), ())


def _k1_body(wt_hbm, w2_hbm, vin, vout, vtin, vtout, *sems):
    sg, ss = sems[:2], sems[2:]
    wid = lax.axis_index("s") * _NC + lax.axis_index("c")
    iota = lax.iota(jnp.int32, 16)
    drow = (iota >> 1)
    dcol = (iota & 1) << 6

    def blk_of(i):
        return jnp.minimum(wid + 32 * i, _NBLK - 1)

    def transpose_scale(src, dst, nebs):
        for eb in range(nebs):
            srccol = _splat(eb * 16) + iota
            dstrow = _splat(eb * 8) + drow
            for fb in (0, 2):
                f0 = _splat(fb * 16)
                f1 = _splat((fb + 1) * 16)

                @pl.loop(0, 16)
                def _d(d):
                    rot = (iota + _splat(d)) & 15
                    r0, r1 = f0 + rot, f1 + rot
                    v0 = plsc.load_gather(src, [r0, srccol])
                    v1 = plsc.load_gather(src, [r1, srccol])
                    plsc.store_scatter(dst, [dstrow, dcol + r0], v0 * _SCALE)
                    plsc.store_scatter(dst, [dstrow, dcol + r1], v1 * _SCALE)

    for b in range(2):
        pltpu.async_copy(
            wt_hbm.at[:, pl.ds(blk_of(b) * 128, 128)], vin.at[b], sg[b])

    @pl.loop(0, _K1_IT, step=2)
    def _it(i):
        for b in range(2):
            pltpu.make_async_copy(
                wt_hbm.at[:, pl.ds(0, 128)], vin.at[b], sg[b]).wait()
            @pl.when(i > 0)
            def _():
                pltpu.make_async_copy(
                    vout.at[b], w2_hbm.at[pl.ds(0, 64)], ss[b]).wait()
            transpose_scale(vin.at[b], vout.at[b], 8)
            @pl.when(i < _K1_IT - 2)
            def _():
                pltpu.async_copy(
                    wt_hbm.at[:, pl.ds(blk_of(i + b + 2) * 128, 128)],
                    vin.at[b], sg[b])
            pltpu.async_copy(
                vout.at[b], w2_hbm.at[pl.ds(blk_of(i + b) * 64, 64)], ss[b])

    for b in range(2):
        pltpu.make_async_copy(
            vout.at[b], w2_hbm.at[pl.ds(0, 64)], ss[b]).wait()

    @pl.when(wid == 0)
    def _tail():
        pltpu.async_copy(
            wt_hbm.at[:, pl.ds(_NBLK * 128, 64)], vtin, sg[0]).wait()
        transpose_scale(vtin, vtout, 4)
        pltpu.async_copy(
            vtout, w2_hbm.at[pl.ds(_NBLK * 64, 32)], ss[0]).wait()


def _k2_body(idx_hbm, w2_hbm, out_hbm, rowidx, halfoff, gb, sb, *sems):
    si, sg, ss = sems[0], sems[1:3], sems[3:]
    wid = lax.axis_index("s") * _NC + lax.axis_index("c")
    pltpu.async_copy(
        idx_hbm.at[pl.ds(wid * _GPW, _GPW)], rowidx, si).wait()

    @pl.loop(0, _GPW, unroll=4)
    def _pre(r):
        for k in range(_G // 16):
            sl = pl.ds(k * 16, 16)
            raw = rowidx[r, sl]
            halfoff[r, sl] = (raw & 1) << 6
            rowidx[r, sl] = raw >> 1

    iota = lax.iota(jnp.int32, 16)
    gg0 = wid * _GPW

    def select_transpose(b, g):
        for jb in range(8):
            rowv = _splat(jb * 16) + iota
            halfv = halfoff[g, pl.ds(jb * 16, 16)]
            for fb in (0, 2):
                h0 = halfv + _splat(fb * 16)
                h1 = halfv + _splat((fb + 1) * 16)
                f0 = _splat(fb * 16)
                f1 = _splat((fb + 1) * 16)

                @pl.loop(0, 16)
                def _d(d):
                    rot = (iota + _splat(d)) & 15
                    v0 = plsc.load_gather(gb.at[b], [rowv, h0 + rot])
                    v1 = plsc.load_gather(gb.at[b], [rowv, h1 + rot])
                    plsc.store_scatter(sb.at[b], [f0 + rot, rowv], v0)
                    plsc.store_scatter(sb.at[b], [f1 + rot, rowv], v1)

    for b in range(2):
        pltpu.async_copy(w2_hbm.at[rowidx.at[b]], gb.at[b], sg[b])

    @pl.loop(0, _GPW, step=2)
    def _grp(g0):
        for b in range(2):
            g = g0 + b
            pltpu.make_async_copy(
                w2_hbm.at[rowidx.at[b]], gb.at[b], sg[b]).wait()
            @pl.when(g0 > 0)
            def _():
                pltpu.make_async_copy(
                    sb.at[b], out_hbm.at[0, :, pl.ds(0, _G)], ss[b]).wait()
            select_transpose(b, g)
            @pl.when(g0 < _GPW - 2)
            def _():
                pltpu.async_copy(w2_hbm.at[rowidx.at[g + 2]], gb.at[b], sg[b])
            gg = gg0 + g
            pltpu.async_copy(
                sb.at[b],
                out_hbm.at[gg >> 7, :, pl.ds((gg & 127) * _G, _G)], ss[b])

    for b in range(2):
        pltpu.make_async_copy(
            sb.at[b], out_hbm.at[0, :, pl.ds(0, _G)], ss[b]).wait()


@jax.jit
def _emb(idx_flat, wt):
    mesh = plsc.VectorSubcoreMesh(core_axis_name="c", subcore_axis_name="s")
    cp = pltpu.CompilerParams(needs_layout_passes=False)
    w2 = pl.kernel(
        _k1_body,
        out_type=jax.ShapeDtypeStruct((_W2ROWS, 128), jnp.float32),
        mesh=mesh,
        compiler_params=cp,
        scratch_types=(
            [pltpu.VMEM((2, _DIM, 128), jnp.float32),
             pltpu.VMEM((2, _DIM, 128), jnp.float32),
             pltpu.VMEM((_DIM, 64), jnp.float32),
             pltpu.VMEM((32, 128), jnp.float32)]
            + [pltpu.SemaphoreType.DMA] * 4
        ),
    )(wt)
    out3 = pl.kernel(
        _k2_body,
        out_type=jax.ShapeDtypeStruct((_SEQ, _DIM, _BATCH), jnp.float32),
        mesh=mesh,
        compiler_params=cp,
        scratch_types=(
            [pltpu.VMEM((_GPW, _G), jnp.int32),
             pltpu.VMEM((_GPW, _G), jnp.int32),
             pltpu.VMEM((2, _G, 128), jnp.float32),
             pltpu.VMEM((2, _DIM, _G), jnp.float32)]
            + [pltpu.SemaphoreType.DMA] * 5
        ),
    )(idx_flat.reshape(_NGRP, _G), w2)
    return out3


def kernel(input, weight):
    idx_flat = input.astype(jnp.int32).T.reshape(_B)
    out3 = _emb(idx_flat, weight.T)
    return out3.transpose(2, 0, 1)


# speedup vs baseline: 2.4186x; 1.0083x over previous
import jax
import jax.numpy as jnp
from jax import lax
from jax.experimental import pallas as pl
from jax.experimental.pallas import tpu as pltpu
from jax.experimental.pallas import tpu_sc as plsc

_NUM_EMB = 1000000
_DIM = 64
_SCALE = float(_DIM) ** 0.5

_NC, _NS = 2, 16
_NW = _NC * _NS
_BATCH, _SEQ = 16384, 50
_B = _BATCH * _SEQ
_NBLK = 7812
_W2ROWS = 500000
_G = 128
_NGRP = _B // _G
_GPW = _NGRP // _NW
_K1_IT = 246


def _splat(x):
    return lax.broadcast_in_dim(x, (16,), ())


def _k1_body(wt_hbm, w2_hbm, vin, vout, vtin, vtout, *sems):
    sg, ss = sems[:3], sems[3:]
    wid = lax.axis_index("s") * _NC + lax.axis_index("c")
    iota = lax.iota(jnp.int32, 16)
    drow = (iota >> 1)
    dcol = (iota & 1) << 6

    def blk_of(i):
        return jnp.minimum(wid + 32 * i, _NBLK - 1)

    def transpose_scale(src, dst, nebs):
        for eb in range(nebs):
            srccol = _splat(eb * 16) + iota
            dstrow = _splat(eb * 8) + drow
            for fb in (0, 2):
                f0 = _splat(fb * 16)
                f1 = _splat((fb + 1) * 16)

                @pl.loop(0, 16)
                def _d(d):
                    rot = (iota + _splat(d)) & 15
                    r0, r1 = f0 + rot, f1 + rot
                    v0 = plsc.load_gather(src, [r0, srccol])
                    v1 = plsc.load_gather(src, [r1, srccol])
                    plsc.store_scatter(dst, [dstrow, dcol + r0], v0 * _SCALE)
                    plsc.store_scatter(dst, [dstrow, dcol + r1], v1 * _SCALE)

    for b in range(3):
        pltpu.async_copy(
            wt_hbm.at[:, pl.ds(blk_of(b) * 128, 128)], vin.at[b], sg[b])

    @pl.loop(0, _K1_IT, step=3)
    def _it(i):
        for b in range(3):
            pltpu.make_async_copy(
                wt_hbm.at[:, pl.ds(0, 128)], vin.at[b], sg[b]).wait()
            @pl.when(i > 0)
            def _():
                pltpu.make_async_copy(
                    vout.at[b], w2_hbm.at[pl.ds(0, 64)], ss[b]).wait()
            transpose_scale(vin.at[b], vout.at[b], 8)
            @pl.when(i < _K1_IT - 3)
            def _():
                pltpu.async_copy(
                    wt_hbm.at[:, pl.ds(blk_of(i + b + 3) * 128, 128)],
                    vin.at[b], sg[b])
            pltpu.async_copy(
                vout.at[b], w2_hbm.at[pl.ds(blk_of(i + b) * 64, 64)], ss[b])

    for b in range(3):
        pltpu.make_async_copy(
            vout.at[b], w2_hbm.at[pl.ds(0, 64)], ss[b]).wait()

    @pl.when(wid == 0)
    def _tail():
        pltpu.async_copy(
            wt_hbm.at[:, pl.ds(_NBLK * 128, 64)], vtin, sg[0]).wait()
        transpose_scale(vtin, vtout, 4)
        pltpu.async_copy(
            vtout, w2_hbm.at[pl.ds(_NBLK * 64, 32)], ss[0]).wait()


def _k2_body(idx_hbm, w2_hbm, out_hbm, rowidx, halfoff, gb, sb, *sems):
    si, sg, ss = sems[0], sems[1:3], sems[3:]
    wid = lax.axis_index("s") * _NC + lax.axis_index("c")
    pltpu.async_copy(
        idx_hbm.at[pl.ds(wid * _GPW, _GPW)], rowidx, si).wait()

    @pl.loop(0, _GPW, unroll=4)
    def _pre(r):
        for k in range(_G // 16):
            sl = pl.ds(k * 16, 16)
            raw = rowidx[r, sl]
            halfoff[r, sl] = (raw & 1) << 6
            rowidx[r, sl] = raw >> 1

    iota = lax.iota(jnp.int32, 16)
    gg0 = wid * _GPW

    def select_transpose(b, g):
        for jb in range(8):
            rowv = _splat(jb * 16) + iota
            halfv = halfoff[g, pl.ds(jb * 16, 16)]
            for fb in (0, 2):
                h0 = halfv + _splat(fb * 16)
                h1 = halfv + _splat((fb + 1) * 16)
                f0 = _splat(fb * 16)
                f1 = _splat((fb + 1) * 16)

                @pl.loop(0, 16)
                def _d(d):
                    rot = (iota + _splat(d)) & 15
                    v0 = plsc.load_gather(gb.at[b], [rowv, h0 + rot])
                    v1 = plsc.load_gather(gb.at[b], [rowv, h1 + rot])
                    plsc.store_scatter(sb.at[b], [f0 + rot, rowv], v0)
                    plsc.store_scatter(sb.at[b], [f1 + rot, rowv], v1)

    for b in range(2):
        pltpu.async_copy(w2_hbm.at[rowidx.at[b]], gb.at[b], sg[b])

    @pl.loop(0, _GPW, step=2)
    def _grp(g0):
        for b in range(2):
            g = g0 + b
            pltpu.make_async_copy(
                w2_hbm.at[rowidx.at[b]], gb.at[b], sg[b]).wait()
            @pl.when(g0 > 0)
            def _():
                pltpu.make_async_copy(
                    sb.at[b], out_hbm.at[0, :, pl.ds(0, _G)], ss[b]).wait()
            select_transpose(b, g)
            @pl.when(g0 < _GPW - 2)
            def _():
                pltpu.async_copy(w2_hbm.at[rowidx.at[g + 2]], gb.at[b], sg[b])
            gg = gg0 + g
            pltpu.async_copy(
                sb.at[b],
                out_hbm.at[gg >> 7, :, pl.ds((gg & 127) * _G, _G)], ss[b])

    for b in range(2):
        pltpu.make_async_copy(
            sb.at[b], out_hbm.at[0, :, pl.ds(0, _G)], ss[b]).wait()


@jax.jit
def _emb(idx_flat, wt):
    mesh = plsc.VectorSubcoreMesh(core_axis_name="c", subcore_axis_name="s")
    cp = pltpu.CompilerParams(needs_layout_passes=False)
    w2 = pl.kernel(
        _k1_body,
        out_type=jax.ShapeDtypeStruct((_W2ROWS, 128), jnp.float32),
        mesh=mesh,
        compiler_params=cp,
        scratch_types=(
            [pltpu.VMEM((3, _DIM, 128), jnp.float32),
             pltpu.VMEM((3, _DIM, 128), jnp.float32),
             pltpu.VMEM((_DIM, 64), jnp.float32),
             pltpu.VMEM((32, 128), jnp.float32)]
            + [pltpu.SemaphoreType.DMA] * 6
        ),
    )(wt)
    out3 = pl.kernel(
        _k2_body,
        out_type=jax.ShapeDtypeStruct((_SEQ, _DIM, _BATCH), jnp.float32),
        mesh=mesh,
        compiler_params=cp,
        scratch_types=(
            [pltpu.VMEM((_GPW, _G), jnp.int32),
             pltpu.VMEM((_GPW, _G), jnp.int32),
             pltpu.VMEM((2, _G, 128), jnp.float32),
             pltpu.VMEM((2, _DIM, _G), jnp.float32)]
            + [pltpu.SemaphoreType.DMA] * 5
        ),
    )(idx_flat.reshape(_NGRP, _G), w2)
    return out3


def kernel(input, weight):
    idx_flat = input.astype(jnp.int32).T.reshape(_B)
    out3 = _emb(idx_flat, weight.T)
    return out3.transpose(2, 0, 1)


# speedup vs baseline: 2.5380x; 1.0493x over previous
import jax
import jax.numpy as jnp
from jax import lax
from jax.experimental import pallas as pl
from jax.experimental.pallas import tpu as pltpu
from jax.experimental.pallas import tpu_sc as plsc

_NUM_EMB = 1000000
_DIM = 64
_SCALE = float(_DIM) ** 0.5

_NC, _NS = 2, 16
_NW = _NC * _NS
_BATCH, _SEQ = 16384, 50
_B = _BATCH * _SEQ
_NBLK = 7812
_W2ROWS = 500000
_G = 128
_NGRP = _B // _G
_GPW = _NGRP // _NW
_K1_IT = 246


def _splat(x):
    return lax.broadcast_in_dim(x, (16,), ())


def _k1_body(wt_hbm, w2_hbm, vin, vout, vtin, vtout, *sems):
    sg, ss = sems[:3], sems[3:]
    wid = lax.axis_index("s") * _NC + lax.axis_index("c")
    iota = lax.iota(jnp.int32, 16)
    drow = (iota >> 1)
    dcol = (iota & 1) << 6

    def blk_of(i):
        return jnp.minimum(wid + 32 * i, _NBLK - 1)

    def transpose_scale(src, dst, nebs):
        for eb in range(nebs):
            srccol = _splat(eb * 16) + iota
            dstrow = _splat(eb * 8) + drow
            for fb in (0, 2):
                f0 = _splat(fb * 16)
                f1 = _splat((fb + 1) * 16)

                @pl.loop(0, 16)
                def _d(d):
                    rot = (iota + _splat(d)) & 15
                    r0, r1 = f0 + rot, f1 + rot
                    v0 = plsc.load_gather(src, [r0, srccol])
                    v1 = plsc.load_gather(src, [r1, srccol])
                    plsc.store_scatter(dst, [dstrow, dcol + r0], v0 * _SCALE)
                    plsc.store_scatter(dst, [dstrow, dcol + r1], v1 * _SCALE)

    for b in range(3):
        pltpu.async_copy(
            wt_hbm.at[:, pl.ds(blk_of(b) * 128, 128)], vin.at[b], sg[b])

    @pl.loop(0, _K1_IT, step=3)
    def _it(i):
        for b in range(3):
            pltpu.make_async_copy(
                wt_hbm.at[:, pl.ds(0, 128)], vin.at[b], sg[b]).wait()
            @pl.when(i > 0)
            def _():
                pltpu.make_async_copy(
                    vout.at[b], w2_hbm.at[pl.ds(0, 64)], ss[b]).wait()
            transpose_scale(vin.at[b], vout.at[b], 8)
            @pl.when(i < _K1_IT - 3)
            def _():
                pltpu.async_copy(
                    wt_hbm.at[:, pl.ds(blk_of(i + b + 3) * 128, 128)],
                    vin.at[b], sg[b])
            pltpu.async_copy(
                vout.at[b], w2_hbm.at[pl.ds(blk_of(i + b) * 64, 64)], ss[b])

    for b in range(3):
        pltpu.make_async_copy(
            vout.at[b], w2_hbm.at[pl.ds(0, 64)], ss[b]).wait()

    @pl.when(wid == 0)
    def _tail():
        pltpu.async_copy(
            wt_hbm.at[:, pl.ds(_NBLK * 128, 64)], vtin, sg[0]).wait()
        transpose_scale(vtin, vtout, 4)
        pltpu.async_copy(
            vtout, w2_hbm.at[pl.ds(_NBLK * 64, 32)], ss[0]).wait()


def _k2_body(idx_hbm, w2_hbm, out_hbm, rowidx, halfoff, gb, sb, *sems):
    si, sg, ss = sems[0], sems[1:3], sems[3:]
    wid = lax.axis_index("s") * _NC + lax.axis_index("c")
    pltpu.async_copy(
        idx_hbm.at[pl.ds(wid * _GPW, _GPW)], rowidx, si).wait()

    @pl.loop(0, _GPW, unroll=4)
    def _pre(r):
        for k in range(_G // 16):
            sl = pl.ds(k * 16, 16)
            raw = rowidx[r, sl]
            halfoff[r, sl] = (raw & 1) << 6
            rowidx[r, sl] = raw >> 1

    iota = lax.iota(jnp.int32, 16)
    gg0 = wid * _GPW

    def select_transpose(b, g):
        for jb in range(8):
            rowv = _splat(jb * 16) + iota
            halfv = halfoff[g, pl.ds(jb * 16, 16)]
            hs = [halfv + _splat(fb * 16) for fb in range(4)]
            fs = [_splat(fb * 16) for fb in range(4)]

            @pl.loop(0, 16)
            def _d(d):
                rot = (iota + _splat(d)) & 15
                vals = [plsc.load_gather(gb.at[b], [rowv, h + rot])
                        for h in hs]
                for f, v in zip(fs, vals):
                    plsc.store_scatter(sb.at[b], [f + rot, rowv], v)

    for b in range(2):
        pltpu.async_copy(w2_hbm.at[rowidx.at[b]], gb.at[b], sg[b])

    @pl.loop(0, _GPW, step=2)
    def _grp(g0):
        for b in range(2):
            g = g0 + b
            pltpu.make_async_copy(
                w2_hbm.at[rowidx.at[b]], gb.at[b], sg[b]).wait()
            @pl.when(g0 > 0)
            def _():
                pltpu.make_async_copy(
                    sb.at[b], out_hbm.at[0, :, pl.ds(0, _G)], ss[b]).wait()
            select_transpose(b, g)
            @pl.when(g0 < _GPW - 2)
            def _():
                pltpu.async_copy(w2_hbm.at[rowidx.at[g + 2]], gb.at[b], sg[b])
            gg = gg0 + g
            pltpu.async_copy(
                sb.at[b],
                out_hbm.at[gg >> 7, :, pl.ds((gg & 127) * _G, _G)], ss[b])

    for b in range(2):
        pltpu.make_async_copy(
            sb.at[b], out_hbm.at[0, :, pl.ds(0, _G)], ss[b]).wait()


@jax.jit
def _emb(idx_flat, wt):
    mesh = plsc.VectorSubcoreMesh(core_axis_name="c", subcore_axis_name="s")
    cp = pltpu.CompilerParams(needs_layout_passes=False)
    w2 = pl.kernel(
        _k1_body,
        out_type=jax.ShapeDtypeStruct((_W2ROWS, 128), jnp.float32),
        mesh=mesh,
        compiler_params=cp,
        scratch_types=(
            [pltpu.VMEM((3, _DIM, 128), jnp.float32),
             pltpu.VMEM((3, _DIM, 128), jnp.float32),
             pltpu.VMEM((_DIM, 64), jnp.float32),
             pltpu.VMEM((32, 128), jnp.float32)]
            + [pltpu.SemaphoreType.DMA] * 6
        ),
    )(wt)
    out3 = pl.kernel(
        _k2_body,
        out_type=jax.ShapeDtypeStruct((_SEQ, _DIM, _BATCH), jnp.float32),
        mesh=mesh,
        compiler_params=cp,
        scratch_types=(
            [pltpu.VMEM((_GPW, _G), jnp.int32),
             pltpu.VMEM((_GPW, _G), jnp.int32),
             pltpu.VMEM((2, _G, 128), jnp.float32),
             pltpu.VMEM((2, _DIM, _G), jnp.float32)]
            + [pltpu.SemaphoreType.DMA] * 5
        ),
    )(idx_flat.reshape(_NGRP, _G), w2)
    return out3


def kernel(input, weight):
    idx_flat = input.astype(jnp.int32).T.reshape(_B)
    out3 = _emb(idx_flat, weight.T)
    return out3.transpose(2, 0, 1)


# speedup vs baseline: 3.1601x; 1.2451x over previous
import jax
import jax.numpy as jnp
from jax import lax
from jax.experimental import pallas as pl
from jax.experimental.pallas import tpu as pltpu
from jax.experimental.pallas import tpu_sc as plsc

_NUM_EMB = 1000000
_DIM = 64
_SCALE = float(_DIM) ** 0.5

_NC, _NS = 2, 16
_NW = _NC * _NS
_BATCH, _SEQ = 16384, 50
_B = _BATCH * _SEQ
_NBLK = 7812
_W2ROWS = 500000
_G = 128
_NGRP = _B // _G
_GPW = _NGRP // _NW
_K1_IT = 246


def _splat(x):
    return lax.broadcast_in_dim(x, (16,), ())


def _k1_body(wt_hbm, w2_hbm, vin, vout, vtin, vtout, *sems):
    sg, ss = sems[:3], sems[3:]
    wid = lax.axis_index("s") * _NC + lax.axis_index("c")
    iota = lax.iota(jnp.int32, 16)
    drow = (iota >> 1)
    dcol = (iota & 1) << 6

    def blk_of(i):
        return jnp.minimum(wid + 32 * i, _NBLK - 1)

    def transpose_scale(src, dst, nebs):
        for eb in range(nebs):
            srccol = _splat(eb * 16) + iota
            dstrow = _splat(eb * 8) + drow
            fs = [_splat(fb * 16) for fb in range(4)]

            @pl.loop(0, 16)
            def _d(d):
                rot = (iota + _splat(d)) & 15
                rows = [f + rot for f in fs]
                vals = [plsc.load_gather(src, [r, srccol]) for r in rows]
                for r, v in zip(rows, vals):
                    plsc.store_scatter(dst, [dstrow, dcol + r], v * _SCALE)

    for b in range(3):
        pltpu.async_copy(
            wt_hbm.at[:, pl.ds(blk_of(b) * 128, 128)], vin.at[b], sg[b])

    @pl.loop(0, _K1_IT, step=3)
    def _it(i):
        for b in range(3):
            pltpu.make_async_copy(
                wt_hbm.at[:, pl.ds(0, 128)], vin.at[b], sg[b]).wait()
            @pl.when(i > 0)
            def _():
                pltpu.make_async_copy(
                    vout.at[b], w2_hbm.at[pl.ds(0, 64)], ss[b]).wait()
            transpose_scale(vin.at[b], vout.at[b], 8)
            @pl.when(i < _K1_IT - 3)
            def _():
                pltpu.async_copy(
                    wt_hbm.at[:, pl.ds(blk_of(i + b + 3) * 128, 128)],
                    vin.at[b], sg[b])
            pltpu.async_copy(
                vout.at[b], w2_hbm.at[pl.ds(blk_of(i + b) * 64, 64)], ss[b])

    for b in range(3):
        pltpu.make_async_copy(
            vout.at[b], w2_hbm.at[pl.ds(0, 64)], ss[b]).wait()

    @pl.when(wid == 0)
    def _tail():
        pltpu.async_copy(
            wt_hbm.at[:, pl.ds(_NBLK * 128, 64)], vtin, sg[0]).wait()
        transpose_scale(vtin, vtout, 4)
        pltpu.async_copy(
            vtout, w2_hbm.at[pl.ds(_NBLK * 64, 32)], ss[0]).wait()


def _k2_body(idx_hbm, w2_hbm, out_hbm, rowidx, halfoff, gb, sb, *sems):
    si, sg, ss = sems[0], sems[1:3], sems[3:]
    wid = lax.axis_index("s") * _NC + lax.axis_index("c")
    pltpu.async_copy(
        idx_hbm.at[pl.ds(wid * _GPW, _GPW)], rowidx, si).wait()

    @pl.loop(0, _GPW, unroll=4)
    def _pre(r):
        for k in range(_G // 16):
            sl = pl.ds(k * 16, 16)
            raw = rowidx[r, sl]
            halfoff[r, sl] = (raw & 1) << 6
            rowidx[r, sl] = raw >> 1

    iota = lax.iota(jnp.int32, 16)
    gg0 = wid * _GPW

    def select_transpose(b, g):
        for jb in range(8):
            rowv = _splat(jb * 16) + iota
            halfv = halfoff[g, pl.ds(jb * 16, 16)]
            hs = [halfv + _splat(fb * 16) for fb in range(4)]
            fs = [_splat(fb * 16) for fb in range(4)]

            @pl.loop(0, 16)
            def _d(d):
                rot = (iota + _splat(d)) & 15
                vals = [plsc.load_gather(gb.at[b], [rowv, h + rot])
                        for h in hs]
                for f, v in zip(fs, vals):
                    plsc.store_scatter(sb.at[b], [f + rot, rowv], v)

    for b in range(2):
        pltpu.async_copy(w2_hbm.at[rowidx.at[b]], gb.at[b], sg[b])

    @pl.loop(0, _GPW, step=2)
    def _grp(g0):
        for b in range(2):
            g = g0 + b
            pltpu.make_async_copy(
                w2_hbm.at[rowidx.at[b]], gb.at[b], sg[b]).wait()
            @pl.when(g0 > 0)
            def _():
                pltpu.make_async_copy(
                    sb.at[b], out_hbm.at[0, :, pl.ds(0, _G)], ss[b]).wait()
            select_transpose(b, g)
            @pl.when(g0 < _GPW - 2)
            def _():
                pltpu.async_copy(w2_hbm.at[rowidx.at[g + 2]], gb.at[b], sg[b])
            gg = gg0 + g
            pltpu.async_copy(
                sb.at[b],
                out_hbm.at[gg >> 7, :, pl.ds((gg & 127) * _G, _G)], ss[b])

    for b in range(2):
        pltpu.make_async_copy(
            sb.at[b], out_hbm.at[0, :, pl.ds(0, _G)], ss[b]).wait()


@jax.jit
def _emb(idx_flat, wt):
    mesh = plsc.VectorSubcoreMesh(core_axis_name="c", subcore_axis_name="s")
    cp = pltpu.CompilerParams(needs_layout_passes=False)
    w2 = pl.kernel(
        _k1_body,
        out_type=jax.ShapeDtypeStruct((_W2ROWS, 128), jnp.float32),
        mesh=mesh,
        compiler_params=cp,
        scratch_types=(
            [pltpu.VMEM((3, _DIM, 128), jnp.float32),
             pltpu.VMEM((3, _DIM, 128), jnp.float32),
             pltpu.VMEM((_DIM, 64), jnp.float32),
             pltpu.VMEM((32, 128), jnp.float32)]
            + [pltpu.SemaphoreType.DMA] * 6
        ),
    )(wt)
    out3 = pl.kernel(
        _k2_body,
        out_type=jax.ShapeDtypeStruct((_SEQ, _DIM, _BATCH), jnp.float32),
        mesh=mesh,
        compiler_params=cp,
        scratch_types=(
            [pltpu.VMEM((_GPW, _G), jnp.int32),
             pltpu.VMEM((_GPW, _G), jnp.int32),
             pltpu.VMEM((2, _G, 128), jnp.float32),
             pltpu.VMEM((2, _DIM, _G), jnp.float32)]
            + [pltpu.SemaphoreType.DMA] * 5
        ),
    )(idx_flat.reshape(_NGRP, _G), w2)
    return out3


def kernel(input, weight):
    idx_flat = input.astype(jnp.int32).T.reshape(_B)
    out3 = _emb(idx_flat, weight.T)
    return out3.transpose(2, 0, 1)
